# baseline (speedup 1.0000x reference)
import functools

import jax
import jax.numpy as jnp
from jax import lax
from jax.experimental import pallas as pl
from jax.experimental.pallas import tpu as pltpu
from jax.experimental.pallas import tpu_sc as plsc

NC = 2
NS = 16
NW = NC * NS
LANES = 16
WIN = 128
WPW = 244
CW = 4
NCHUNK = WPW // CW
SENT = 0x7F000000


@functools.lru_cache(maxsize=None)
def _build_gather(n_tok: int, v: int, d: int, seq: int = 4096):
    nvec_tok = n_tok // LANES
    vpad = ((v + WIN - 1) // WIN) * WIN
    last_win = (v - 1) // WIN
    base_win = NW * WPW
    mesh = plsc.VectorSubcoreMesh(core_axis_name="c", subcore_axis_name="s")

    @functools.partial(
        pl.kernel,
        mesh=mesh,
        out_type=jax.ShapeDtypeStruct((n_tok * d,), jnp.float32),
        scratch_types=[
            pltpu.VMEM((n_tok,), jnp.int32),
            pltpu.VMEM((n_tok + LANES,), jnp.int32),
            pltpu.VMEM((n_tok,), jnp.int32),
            pltpu.VMEM((2, d, CW * WIN), jnp.float32),
            pltpu.VMEM((LANES, d), jnp.float32),
            pltpu.VMEM((LANES, d), jnp.float32),
            pltpu.SemaphoreType.DMA,
            pltpu.SemaphoreType.DMA,
            pltpu.SemaphoreType.DMA,
            pltpu.SemaphoreType.DMA,
            pltpu.SemaphoreType.DMA,
        ],
        compiler_params=pltpu.CompilerParams(needs_layout_passes=False),
    )
    def gather(idx_hbm, tok_hbm, pos_hbm, out_hbm, idx_v, pk_v, ch_v, blk_v,
               row_v, ps_v, semA, semB, isem, wsem, psem):
        wid = lax.axis_index("s") * NC + lax.axis_index("c")
        wlo = wid * WPW
        ci = lax.iota(jnp.int32, LANES)
        lane0 = ci == 0

        pltpu.async_copy(idx_hbm, idx_v, isem).wait()

        swin = base_win + wid

        def scan(i, off):
            vec = idx_v[pl.ds(i * LANES, LANES)]
            win = jnp.right_shift(vec, 7)
            rel = win - wlo
            m_main = (rel >= 0) & (rel < WPW)
            m_spec = win == swin
            rel = jnp.where(m_spec, WPW, rel)
            m = m_main | m_spec
            lane = jnp.bitwise_and(vec, WIN - 1)
            pos = i * LANES + ci
            packed = (
                jnp.left_shift(rel, 21)
                | jnp.left_shift(lane, 14)
                | pos
            )
            plsc.store_compressed(pk_v.at[pl.ds(off, LANES)], packed, mask=m)
            cnt = plsc.all_reduce_population_count(m)
            return off + cnt[0]

        nhit = lax.fori_loop(0, nvec_tok, scan, 0)
        pk_v[pl.ds(nhit, LANES)] = jnp.full((LANES,), SENT, jnp.int32)
        nv = lax.div(nhit + LANES - 1, LANES)

        def match_chunk(c, buf):
            def mbody(j2, moff):
                sv = pk_v[pl.ds(j2 * LANES, LANES)]
                m = jnp.right_shift(sv, 23) == c
                plsc.store_compressed(ch_v.at[pl.ds(moff, LANES)], sv, mask=m)
                cnt = plsc.all_reduce_population_count(m)
                return moff + cnt[0]

            mcnt = lax.fori_loop(0, nv, mbody, 0)

            def ebody(j2, carry):
                sv = ch_v[pl.ds(j2 * LANES, LANES)]
                rem = mcnt - j2 * LANES
                for j in range(LANES):
                    @pl.when(j < rem)
                    def _():
                        p = sv[j]
                        pos = jnp.bitwise_and(p, (1 << 14) - 1)
                        srow = jnp.bitwise_and(pos, seq - 1)
                        pltpu.async_copy(
                            pos_hbm.at[pl.ds(srow * d, d)], ps_v.at[j], psem
                        )
                for j in range(LANES):
                    @pl.when(j < rem)
                    def _():
                        p = sv[j]
                        relw = jnp.right_shift(p, 21)
                        wc = jnp.bitwise_and(relw, CW - 1)
                        lane = jnp.bitwise_and(jnp.right_shift(p, 14), WIN - 1)
                        col = jnp.full((LANES,), wc * WIN + lane, jnp.int32)
                        bi = jnp.full((LANES,), buf, jnp.int32)
                        for cc in range(d // LANES):
                            vals = plsc.load_gather(
                                blk_v, [bi, cc * LANES + ci, col]
                            )
                            row_v[j, pl.ds(cc * LANES, LANES)] = vals

                nw = jnp.maximum(jnp.minimum(rem, LANES), 0)

                def pdrain(i, c2):
                    pltpu.make_async_copy(
                        pos_hbm.at[pl.ds(0, d)], ps_v.at[0], psem
                    ).wait()
                    return c2

                lax.fori_loop(0, nw, pdrain, 0)
                for j in range(LANES):
                    @pl.when(j < rem)
                    def _():
                        p = sv[j]
                        pos = jnp.bitwise_and(p, (1 << 14) - 1)
                        for cc in range(d // LANES):
                            sl = pl.ds(cc * LANES, LANES)
                            row_v[j, sl] = row_v[j, sl] + ps_v[j, sl]
                        pltpu.async_copy(
                            row_v.at[j],
                            out_hbm.at[pl.ds(pos * d, d)],
                            wsem,
                        )

                def wdrain(i, c2):
                    pltpu.make_async_copy(
                        row_v.at[0], out_hbm.at[pl.ds(0, d)], wsem
                    ).wait()
                    return c2

                lax.fori_loop(0, nw, wdrain, 0)
                return carry

            nmv = lax.div(mcnt + LANES - 1, LANES)
            lax.fori_loop(0, nmv, ebody, 0)

        def fire(c, buf, sem):
            start = (wlo + c * CW) * WIN
            pltpu.async_copy(
                tok_hbm.at[:, pl.ds(start, CW * WIN)], blk_v.at[buf], sem
            )

        def drain(sem):
            pltpu.make_async_copy(
                tok_hbm.at[:, pl.ds(0, CW * WIN)], blk_v.at[0], sem
            ).wait()

        fire(0, 0, semA)

        def pair_body(k, carry):
            c0 = k * 2
            fire(c0 + 1, 1, semB)
            drain(semA)
            match_chunk(c0, 0)
            fire(c0 + 2, 0, semA)
            drain(semB)
            match_chunk(c0 + 1, 1)
            return carry

        lax.fori_loop(0, (NCHUNK - 1) // 2, pair_body, 0)
        drain(semA)
        match_chunk(NCHUNK - 1, 0)

        sw = jnp.minimum(swin, last_win) * WIN
        sw = pl.multiple_of(sw, WIN)
        pltpu.async_copy(
            tok_hbm.at[:, pl.ds(sw, WIN)],
            blk_v.at[1, :, pl.ds(0, WIN)],
            semB,
        )
        pltpu.make_async_copy(
            tok_hbm.at[:, pl.ds(0, WIN)],
            blk_v.at[1, :, pl.ds(0, WIN)],
            semB,
        ).wait()
        match_chunk(NCHUNK, 1)

    return gather


@functools.lru_cache(maxsize=None)
def _build_addpos(n_tok: int, seq: int, d: int):
    bpw = n_tok // NW
    mesh = plsc.VectorSubcoreMesh(core_axis_name="c", subcore_axis_name="s")

    @functools.partial(
        pl.kernel,
        mesh=mesh,
        out_type=jax.ShapeDtypeStruct((n_tok * d,), jnp.float32),
        scratch_types=[
            pltpu.VMEM((bpw * d,), jnp.float32),
            pltpu.VMEM((bpw * d,), jnp.float32),
            pltpu.SemaphoreType.DMA,
            pltpu.SemaphoreType.DMA,
        ],
        compiler_params=pltpu.CompilerParams(needs_layout_passes=False),
    )
    def addpos(rows_hbm, pos_hbm, out_hbm, rows_v, pos_v, sem, psem):
        wid = lax.axis_index("s") * NC + lax.axis_index("c")
        base = wid * bpw
        s0 = lax.rem(base, seq)
        cpa = pltpu.async_copy(
            rows_hbm.at[pl.ds(base * d, bpw * d)], rows_v, sem
        )
        cpb = pltpu.async_copy(
            pos_hbm.at[pl.ds(s0 * d, bpw * d)], pos_v, psem
        )
        cpa.wait()
        cpb.wait()

        def body(i, carry):
            sl = pl.ds(i * LANES, LANES)
            rows_v[sl] = rows_v[sl] + pos_v[sl]
            return carry

        lax.fori_loop(0, bpw * d // LANES, body, 0)
        pltpu.sync_copy(rows_v, out_hbm.at[pl.ds(base * d, bpw * d)])

    return addpos


def kernel(inputs, token_table, pos_table):
    b, s = inputs.shape
    v, d = token_table.shape
    n_tok = b * s
    idx = inputs.reshape(-1).astype(jnp.int32)
    out = _build_gather(n_tok, v, d, s)(
        idx, token_table.T, pos_table.reshape(-1)
    )
    return out.reshape(b, s, d)


# speedup vs baseline: 1.2445x; 1.2445x over previous
import functools

import jax
import jax.numpy as jnp
from jax import lax
from jax.experimental import pallas as pl
from jax.experimental.pallas import tpu as pltpu
from jax.experimental.pallas import tpu_sc as plsc

NC = 2
NS = 16
NW = NC * NS
LANES = 16
WIN = 128
WPW = 244
CW = 4
NCHUNK = WPW // CW
SENT = 0x7F000000


@functools.lru_cache(maxsize=None)
def _build_gather(n_tok: int, v: int, d: int):
    nvec_tok = n_tok // LANES
    vpad = ((v + WIN - 1) // WIN) * WIN
    last_win = (v - 1) // WIN
    base_win = NW * WPW
    mesh = plsc.VectorSubcoreMesh(core_axis_name="c", subcore_axis_name="s")

    @functools.partial(
        pl.kernel,
        mesh=mesh,
        out_type=jax.ShapeDtypeStruct((n_tok * d,), jnp.float32),
        scratch_types=[
            pltpu.VMEM((n_tok,), jnp.int32),
            pltpu.VMEM((n_tok + LANES,), jnp.int32),
            pltpu.VMEM((n_tok,), jnp.int32),
            pltpu.VMEM((2, d, CW * WIN), jnp.float32),
            pltpu.VMEM((LANES, d), jnp.float32),
            pltpu.SemaphoreType.DMA,
            pltpu.SemaphoreType.DMA,
            pltpu.SemaphoreType.DMA,
            pltpu.SemaphoreType.DMA,
        ],
        compiler_params=pltpu.CompilerParams(needs_layout_passes=False),
    )
    def gather(idx_hbm, tok_hbm, out_hbm, idx_v, pk_v, ch_v, blk_v, row_v,
               semA, semB, isem, wsem):
        wid = lax.axis_index("s") * NC + lax.axis_index("c")
        wlo = wid * WPW
        ci = lax.iota(jnp.int32, LANES)
        lane0 = ci == 0

        pltpu.async_copy(idx_hbm, idx_v, isem).wait()

        swin = base_win + wid

        def scan(i, off):
            vec = idx_v[pl.ds(i * LANES, LANES)]
            win = jnp.right_shift(vec, 7)
            rel = win - wlo
            m_main = (rel >= 0) & (rel < WPW)
            m_spec = win == swin
            rel = jnp.where(m_spec, WPW, rel)
            m = m_main | m_spec
            lane = jnp.bitwise_and(vec, WIN - 1)
            pos = i * LANES + ci
            packed = (
                jnp.left_shift(rel, 21)
                | jnp.left_shift(lane, 14)
                | pos
            )
            plsc.store_compressed(pk_v.at[pl.ds(off, LANES)], packed, mask=m)
            cnt = plsc.all_reduce_population_count(m)
            return off + cnt[0]

        nhit = lax.fori_loop(0, nvec_tok, scan, 0)
        pk_v[pl.ds(nhit, LANES)] = jnp.full((LANES,), SENT, jnp.int32)
        nv = lax.div(nhit + LANES - 1, LANES)

        def match_chunk(c, buf):
            def mbody(j2, moff):
                sv = pk_v[pl.ds(j2 * LANES, LANES)]
                m = jnp.right_shift(sv, 23) == c
                plsc.store_compressed(ch_v.at[pl.ds(moff, LANES)], sv, mask=m)
                cnt = plsc.all_reduce_population_count(m)
                return moff + cnt[0]

            mcnt = lax.fori_loop(0, nv, mbody, 0)

            def ebody(j2, carry):
                sv = ch_v[pl.ds(j2 * LANES, LANES)]
                rem = mcnt - j2 * LANES
                for j in range(LANES):
                    @pl.when(j < rem)
                    def _():
                        p = sv[j]
                        relw = jnp.right_shift(p, 21)
                        wc = jnp.bitwise_and(relw, CW - 1)
                        lane = jnp.bitwise_and(jnp.right_shift(p, 14), WIN - 1)
                        pos = jnp.bitwise_and(p, (1 << 14) - 1)
                        col = jnp.full((LANES,), wc * WIN + lane, jnp.int32)
                        bi = jnp.full((LANES,), buf, jnp.int32)
                        for cc in range(d // LANES):
                            vals = plsc.load_gather(
                                blk_v, [bi, cc * LANES + ci, col]
                            )
                            row_v[j, pl.ds(cc * LANES, LANES)] = vals
                        pltpu.async_copy(
                            row_v.at[j],
                            out_hbm.at[pl.ds(pos * d, d)],
                            wsem,
                        )

                def wdrain(i, c2):
                    pltpu.make_async_copy(
                        row_v.at[0], out_hbm.at[pl.ds(0, d)], wsem
                    ).wait()
                    return c2

                nw = jnp.minimum(rem, LANES)
                nw = jnp.maximum(nw, 0)
                lax.fori_loop(0, nw, wdrain, 0)
                return carry

            nmv = lax.div(mcnt + LANES - 1, LANES)
            lax.fori_loop(0, nmv, ebody, 0)

        def fire(c, buf, sem):
            start = (wlo + c * CW) * WIN
            pltpu.async_copy(
                tok_hbm.at[:, pl.ds(start, CW * WIN)], blk_v.at[buf], sem
            )

        def drain(sem):
            pltpu.make_async_copy(
                tok_hbm.at[:, pl.ds(0, CW * WIN)], blk_v.at[0], sem
            ).wait()

        fire(0, 0, semA)

        def pair_body(k, carry):
            c0 = k * 2
            fire(c0 + 1, 1, semB)
            drain(semA)
            match_chunk(c0, 0)
            fire(c0 + 2, 0, semA)
            drain(semB)
            match_chunk(c0 + 1, 1)
            return carry

        lax.fori_loop(0, (NCHUNK - 1) // 2, pair_body, 0)
        drain(semA)
        match_chunk(NCHUNK - 1, 0)

        sw = jnp.minimum(swin, last_win) * WIN
        sw = pl.multiple_of(sw, WIN)
        pltpu.async_copy(
            tok_hbm.at[:, pl.ds(sw, WIN)],
            blk_v.at[1, :, pl.ds(0, WIN)],
            semB,
        )
        pltpu.make_async_copy(
            tok_hbm.at[:, pl.ds(0, WIN)],
            blk_v.at[1, :, pl.ds(0, WIN)],
            semB,
        ).wait()
        match_chunk(NCHUNK, 1)

    return gather


@functools.lru_cache(maxsize=None)
def _build_addpos(n_tok: int, seq: int, d: int):
    bpw = n_tok // NW
    mesh = plsc.VectorSubcoreMesh(core_axis_name="c", subcore_axis_name="s")

    @functools.partial(
        pl.kernel,
        mesh=mesh,
        out_type=jax.ShapeDtypeStruct((n_tok * d,), jnp.float32),
        scratch_types=[
            pltpu.VMEM((bpw * d,), jnp.float32),
            pltpu.VMEM((bpw * d,), jnp.float32),
            pltpu.SemaphoreType.DMA,
            pltpu.SemaphoreType.DMA,
        ],
        compiler_params=pltpu.CompilerParams(needs_layout_passes=False),
    )
    def addpos(rows_hbm, pos_hbm, out_hbm, rows_v, pos_v, sem, psem):
        wid = lax.axis_index("s") * NC + lax.axis_index("c")
        base = wid * bpw
        s0 = lax.rem(base, seq)
        cpa = pltpu.async_copy(
            rows_hbm.at[pl.ds(base * d, bpw * d)], rows_v, sem
        )
        cpb = pltpu.async_copy(
            pos_hbm.at[pl.ds(s0 * d, bpw * d)], pos_v, psem
        )
        cpa.wait()
        cpb.wait()

        def body(i, carry):
            sl = pl.ds(i * LANES, LANES)
            rows_v[sl] = rows_v[sl] + pos_v[sl]
            return carry

        lax.fori_loop(0, bpw * d // LANES, body, 0)
        pltpu.sync_copy(rows_v, out_hbm.at[pl.ds(base * d, bpw * d)])

    return addpos


def kernel(inputs, token_table, pos_table):
    b, s = inputs.shape
    v, d = token_table.shape
    n_tok = b * s
    idx = inputs.reshape(-1).astype(jnp.int32)
    rows = _build_gather(n_tok, v, d)(idx, token_table.T)
    out = _build_addpos(n_tok, s, d)(rows, pos_table.reshape(-1))
    return out.reshape(b, s, d)
